# Initial kernel scaffold; baseline (speedup 1.0000x reference)
import jax
import jax.numpy as jnp
from jax.experimental import pallas as pl


def kernel(input_ids, word_embeddings):
    raise NotImplementedError("write your pallas kernel here")


# speedup vs baseline: 9.2669x; 9.2669x over previous
import functools

import jax
import jax.numpy as jnp
from jax import lax
from jax.experimental import pallas as pl
from jax.experimental.pallas import tpu as pltpu
from jax.experimental.pallas import tpu_sc as plsc

_HIDDEN = 128
_CHUNK = 128
_NBUF = 4
_NC, _NS = 2, 16
_NW = _NC * _NS


def _run(idx2d, table):
    n_chunks = idx2d.shape[0]
    steps = n_chunks // _NW
    outer = steps // _NBUF

    mesh = plsc.VectorSubcoreMesh(core_axis_name="c", subcore_axis_name="s")

    @functools.partial(
        pl.kernel,
        mesh=mesh,
        out_type=jax.ShapeDtypeStruct((n_chunks * _CHUNK, _HIDDEN), jnp.float32),
        scratch_types=(
            [pltpu.VMEM((steps, _CHUNK), jnp.int32),
             pltpu.VMEM((_NBUF, _CHUNK, _HIDDEN), jnp.float32)]
            + [pltpu.SemaphoreType.DMA] * (2 * _NBUF)
        ),
    )
    def k(idx_hbm, table_hbm, out_hbm, idx_v, rows_v, *sems):
        gsem = sems[:_NBUF]
        osem = sems[_NBUF:]
        wid = lax.axis_index("s") * _NC + lax.axis_index("c")
        chunk0 = wid * steps

        pltpu.sync_copy(idx_hbm.at[pl.ds(chunk0, steps)], idx_v)

        def gather_start(g, b):
            pltpu.async_copy(table_hbm.at[idx_v.at[g]], rows_v.at[b], gsem[b])

        def gather_wait(b):
            pltpu.make_async_copy(
                table_hbm.at[idx_v.at[0]], rows_v.at[b], gsem[b]).wait()

        def out_start(g, b):
            pltpu.async_copy(
                rows_v.at[b],
                out_hbm.at[pl.ds((chunk0 + g) * _CHUNK, _CHUNK)], osem[b])

        def out_wait(b):
            pltpu.make_async_copy(
                rows_v.at[b], out_hbm.at[pl.ds(0, _CHUNK)], osem[b]).wait()

        for b in range(_NBUF):
            gather_start(b, b)

        def body(o, carry):
            for b in range(_NBUF):
                g = o * _NBUF + b
                gather_wait(b)
                out_start(g, b)
                nxt = g + _NBUF

                @pl.when(nxt < steps)
                def _():
                    out_wait(b)
                    gather_start(nxt, b)
            return carry

        lax.fori_loop(0, outer, body, 0)
        for b in range(_NBUF):
            out_wait(b)

    return k(idx2d, table)


def kernel(input_ids, word_embeddings):
    b, l = input_ids.shape
    n = b * l
    idx2d = input_ids.astype(jnp.int32).reshape(n // _CHUNK, _CHUNK)
    out = _run(idx2d, word_embeddings)
    return out.reshape(b, l, _HIDDEN)


# speedup vs baseline: 9.2805x; 1.0015x over previous
import functools

import jax
import jax.numpy as jnp
from jax import lax
from jax.experimental import pallas as pl
from jax.experimental.pallas import tpu as pltpu
from jax.experimental.pallas import tpu_sc as plsc

_HIDDEN = 128
_CHUNK = 128
_NBUF = 5
_NC, _NS = 2, 16
_NW = _NC * _NS


def _run(idx2d, table):
    n_chunks = idx2d.shape[0]
    steps = n_chunks // _NW
    outer = steps // _NBUF

    mesh = plsc.VectorSubcoreMesh(core_axis_name="c", subcore_axis_name="s")

    @functools.partial(
        pl.kernel,
        mesh=mesh,
        out_type=jax.ShapeDtypeStruct((n_chunks * _CHUNK, _HIDDEN), jnp.float32),
        scratch_types=(
            [pltpu.VMEM((steps, _CHUNK), jnp.int32),
             pltpu.VMEM((_NBUF, _CHUNK, _HIDDEN), jnp.float32)]
            + [pltpu.SemaphoreType.DMA] * (2 * _NBUF)
        ),
    )
    def k(idx_hbm, table_hbm, out_hbm, idx_v, rows_v, *sems):
        gsem = sems[:_NBUF]
        osem = sems[_NBUF:]
        wid = lax.axis_index("s") * _NC + lax.axis_index("c")
        chunk0 = wid * steps

        pltpu.sync_copy(idx_hbm.at[pl.ds(chunk0, steps)], idx_v)

        def gather_start(g, b):
            pltpu.async_copy(table_hbm.at[idx_v.at[g]], rows_v.at[b], gsem[b])

        def gather_wait(b):
            pltpu.make_async_copy(
                table_hbm.at[idx_v.at[0]], rows_v.at[b], gsem[b]).wait()

        def out_start(g, b):
            pltpu.async_copy(
                rows_v.at[b],
                out_hbm.at[pl.ds((chunk0 + g) * _CHUNK, _CHUNK)], osem[b])

        def out_wait(b):
            pltpu.make_async_copy(
                rows_v.at[b], out_hbm.at[pl.ds(0, _CHUNK)], osem[b]).wait()

        for b in range(_NBUF):
            gather_start(b, b)

        def body(o, carry):
            for b in range(_NBUF):
                g = o * _NBUF + b
                gather_wait(b)
                out_start(g, b)
                nxt = g + _NBUF

                @pl.when(nxt < steps)
                def _():
                    out_wait(b)
                    gather_start(nxt, b)
            return carry

        lax.fori_loop(0, outer, body, 0)
        for b in range(_NBUF):
            out_wait(b)

    return k(idx2d, table)


def kernel(input_ids, word_embeddings):
    b, l = input_ids.shape
    n = b * l
    idx2d = input_ids.astype(jnp.int32).reshape(n // _CHUNK, _CHUNK)
    out = _run(idx2d, word_embeddings)
    return out.reshape(b, l, _HIDDEN)


# speedup vs baseline: 9.2944x; 1.0015x over previous
import functools

import jax
import jax.numpy as jnp
from jax import lax
from jax.experimental import pallas as pl
from jax.experimental.pallas import tpu as pltpu
from jax.experimental.pallas import tpu_sc as plsc

_HIDDEN = 128
_IW = 128
_HALF = 2
_CHUNK = _IW * _HALF
_NBUF = 3
_NC, _NS = 2, 16
_NW = _NC * _NS


def _run(idx2d, table):
    n_iw = idx2d.shape[0]
    n_chunks = n_iw // _HALF
    steps = n_chunks // _NW
    outer = (steps + _NBUF - 1) // _NBUF

    mesh = plsc.VectorSubcoreMesh(core_axis_name="c", subcore_axis_name="s")

    @functools.partial(
        pl.kernel,
        mesh=mesh,
        out_type=jax.ShapeDtypeStruct((n_chunks, _HALF, _IW, _HIDDEN),
                                      jnp.float32),
        scratch_types=(
            [pltpu.VMEM((steps * _HALF, _IW), jnp.int32),
             pltpu.VMEM((_NBUF, _HALF, _IW, _HIDDEN), jnp.float32)]
            + [pltpu.SemaphoreType.DMA] * (2 * _NBUF)
        ),
    )
    def k(idx_hbm, table_hbm, out_hbm, idx_v, rows_v, *sems):
        gsem = sems[:_NBUF]
        osem = sems[_NBUF:]
        wid = lax.axis_index("s") * _NC + lax.axis_index("c")
        chunk0 = wid * steps

        pltpu.sync_copy(idx_hbm.at[pl.ds(chunk0 * _HALF, steps * _HALF)],
                        idx_v)

        def gather_start(g, b):
            for h in range(_HALF):
                pltpu.async_copy(table_hbm.at[idx_v.at[g * _HALF + h]],
                                 rows_v.at[b, h], gsem[b])

        def gather_wait(b):
            pltpu.make_async_copy(
                out_hbm.at[chunk0], rows_v.at[b], gsem[b]).wait()

        def out_start(g, b):
            pltpu.async_copy(rows_v.at[b], out_hbm.at[chunk0 + g], osem[b])

        def out_wait(b):
            pltpu.make_async_copy(
                rows_v.at[b], out_hbm.at[chunk0], osem[b]).wait()

        for b in range(_NBUF):
            gather_start(b, b)

        def body(o, carry):
            for b in range(_NBUF):
                g = o * _NBUF + b

                @pl.when(g < steps)
                def _():
                    gather_wait(b)
                    out_start(g, b)

                nxt = g + _NBUF

                @pl.when(nxt < steps)
                def _():
                    out_wait(b)
                    gather_start(nxt, b)
            return carry

        lax.fori_loop(0, outer, body, 0)
        for b in range(_NBUF):
            out_wait(b)

    return k(idx2d, table)


def kernel(input_ids, word_embeddings):
    b, l = input_ids.shape
    n = b * l
    idx2d = input_ids.astype(jnp.int32).reshape(n // _IW, _IW)
    out = _run(idx2d, word_embeddings)
    return out.reshape(b, l, _HIDDEN)
